# Initial kernel scaffold; baseline (speedup 1.0000x reference)
import jax
import jax.numpy as jnp
from jax.experimental import pallas as pl


def kernel(queries, keys):
    raise NotImplementedError("write your pallas kernel here")


# speedup vs baseline: 1.1785x; 1.1785x over previous
import functools

import jax
import jax.numpy as jnp
from jax.experimental import pallas as pl
from jax.experimental.pallas import tpu as pltpu


_QB = 1024
_KB = 2048


def _nn_body(q_ref, k_ref, qsq_ref, ksq_ref, dist_ref, idx_ref, *, kb):
    j = pl.program_id(1)
    q = q_ref[...]
    k = k_ref[...]
    cross = jax.lax.dot_general(
        q, k, (((1,), (1,)), ((), ())),
        preferred_element_type=jnp.float32)
    d2 = (qsq_ref[...] + ksq_ref[...]) - 2.0 * cross

    local_min = jnp.min(d2, axis=1)
    iota = jax.lax.broadcasted_iota(jnp.int32, d2.shape, 1)
    local_arg = jnp.min(
        jnp.where(d2 == local_min[:, None], iota, jnp.int32(2**31 - 1)),
        axis=1) + j * kb

    @pl.when(j == 0)
    def _init():
        dist_ref[...] = local_min[:, None]
        idx_ref[...] = local_arg[:, None]

    @pl.when(j > 0)
    def _update():
        prev = dist_ref[:, 0]
        better = local_min < prev
        dist_ref[...] = jnp.where(better, local_min, prev)[:, None]
        idx_ref[...] = jnp.where(better, local_arg, idx_ref[:, 0])[:, None]


@jax.jit
def kernel(queries, keys):
    q_count, d = queries.shape
    k_count = keys.shape[0]
    k_pad = ((k_count + _KB - 1) // _KB) * _KB
    keys_p = jnp.pad(keys, ((0, k_pad - k_count), (0, 0)),
                     constant_values=1e4)
    q_sq = jnp.sum(queries * queries, axis=-1, keepdims=True)
    k_sq = jnp.sum(keys_p * keys_p, axis=-1)[None, :]

    grid = (q_count // _QB, k_pad // _KB)
    dist, idx = pl.pallas_call(
        functools.partial(_nn_body, kb=_KB),
        grid=grid,
        in_specs=[
            pl.BlockSpec((_QB, d), lambda i, j: (i, 0)),
            pl.BlockSpec((_KB, d), lambda i, j: (j, 0)),
            pl.BlockSpec((_QB, 1), lambda i, j: (i, 0)),
            pl.BlockSpec((1, _KB), lambda i, j: (0, j)),
        ],
        out_specs=[
            pl.BlockSpec((_QB, 1), lambda i, j: (i, 0)),
            pl.BlockSpec((_QB, 1), lambda i, j: (i, 0)),
        ],
        out_shape=[
            jax.ShapeDtypeStruct((q_count, 1), jnp.float32),
            jax.ShapeDtypeStruct((q_count, 1), jnp.int32),
        ],
        compiler_params=pltpu.CompilerParams(
            dimension_semantics=("parallel", "arbitrary")),
    )(queries, keys_p, q_sq, k_sq)
    return dist, idx.astype(jnp.int64)


# speedup vs baseline: 1.5036x; 1.2758x over previous
import functools

import jax
import jax.numpy as jnp
from jax.experimental import pallas as pl
from jax.experimental.pallas import tpu as pltpu


_QB = 1024
_KB = 2048


def _nn_body(q_ref, k_ref, qsq_ref, ksq_ref, dist_ref, idx_ref, *, kb):
    j = pl.program_id(1)
    q = q_ref[...]
    k2 = k_ref[...]
    qb = q.shape[0]
    cross2 = jax.lax.dot_general(
        q, k2, (((1,), (1,)), ((), ())),
        preferred_element_type=jnp.float32)
    qsq = jnp.broadcast_to(qsq_ref[...], (qb, 128))
    ksq = ksq_ref[...]

    run_min = (qsq + ksq[:, 0:128]) - cross2[:, 0:128]
    run_chunk = jnp.zeros((qb, 128), jnp.int32)
    for c in range(1, kb // 128):
        dc = (qsq + ksq[:, c * 128:(c + 1) * 128]) \
            - cross2[:, c * 128:(c + 1) * 128]
        pred = dc < run_min
        run_min = jnp.where(pred, dc, run_min)
        run_chunk = jnp.where(pred, jnp.int32(c), run_chunk)

    lane = jax.lax.broadcasted_iota(jnp.int32, (qb, 128), 1)
    gidx = run_chunk * 128 + lane + j * kb
    local_min = jnp.min(run_min, axis=1)
    local_arg = jnp.min(
        jnp.where(run_min == local_min[:, None], gidx, jnp.int32(2**31 - 1)),
        axis=1)

    @pl.when(j == 0)
    def _init():
        dist_ref[...] = local_min[:, None]
        idx_ref[...] = local_arg[:, None]

    @pl.when(j > 0)
    def _update():
        prev = dist_ref[:, 0]
        better = local_min < prev
        dist_ref[...] = jnp.where(better, local_min, prev)[:, None]
        idx_ref[...] = jnp.where(better, local_arg, idx_ref[:, 0])[:, None]


@jax.jit
def kernel(queries, keys):
    q_count, d = queries.shape
    k_count = keys.shape[0]
    k_pad = ((k_count + _KB - 1) // _KB) * _KB
    keys_p = jnp.pad(keys, ((0, k_pad - k_count), (0, 0)),
                     constant_values=1e4)
    q_sq = jnp.sum(queries * queries, axis=-1, keepdims=True)
    k_sq = jnp.sum(keys_p * keys_p, axis=-1)[None, :]
    keys_p = keys_p * 2.0

    grid = (q_count // _QB, k_pad // _KB)
    dist, idx = pl.pallas_call(
        functools.partial(_nn_body, kb=_KB),
        grid=grid,
        in_specs=[
            pl.BlockSpec((_QB, d), lambda i, j: (i, 0)),
            pl.BlockSpec((_KB, d), lambda i, j: (j, 0)),
            pl.BlockSpec((_QB, 1), lambda i, j: (i, 0)),
            pl.BlockSpec((1, _KB), lambda i, j: (0, j)),
        ],
        out_specs=[
            pl.BlockSpec((_QB, 1), lambda i, j: (i, 0)),
            pl.BlockSpec((_QB, 1), lambda i, j: (i, 0)),
        ],
        out_shape=[
            jax.ShapeDtypeStruct((q_count, 1), jnp.float32),
            jax.ShapeDtypeStruct((q_count, 1), jnp.int32),
        ],
        compiler_params=pltpu.CompilerParams(
            dimension_semantics=("parallel", "arbitrary")),
    )(queries, keys_p, q_sq, k_sq)
    return dist, idx.astype(jnp.int64)


# speedup vs baseline: 1.8072x; 1.2019x over previous
import functools

import jax
import jax.numpy as jnp
from jax.experimental import pallas as pl
from jax.experimental.pallas import tpu as pltpu


_QB = 2048
_KB = 6400


def _nn_body(q_ref, k_ref, qsq_ref, ksq_ref, dist_ref, idx_ref, *, kb):
    j = pl.program_id(1)
    q = q_ref[...]
    k2 = k_ref[...]
    qb = q.shape[0]
    cross2 = jax.lax.dot_general(
        q, k2, (((1,), (1,)), ((), ())),
        preferred_element_type=jnp.float32)
    qsq = jnp.broadcast_to(qsq_ref[...], (qb, 128))
    ksq = ksq_ref[...]

    run_min = (qsq + ksq[:, 0:128]) - cross2[:, 0:128]
    run_chunk = jnp.zeros((qb, 128), jnp.int32)
    for c in range(1, kb // 128):
        dc = (qsq + ksq[:, c * 128:(c + 1) * 128]) \
            - cross2[:, c * 128:(c + 1) * 128]
        pred = dc < run_min
        run_min = jnp.where(pred, dc, run_min)
        run_chunk = jnp.where(pred, jnp.int32(c), run_chunk)

    lane = jax.lax.broadcasted_iota(jnp.int32, (qb, 128), 1)
    gidx = run_chunk * 128 + lane + j * kb
    local_min = jnp.min(run_min, axis=1)
    local_arg = jnp.min(
        jnp.where(run_min == local_min[:, None], gidx, jnp.int32(2**31 - 1)),
        axis=1)

    @pl.when(j == 0)
    def _init():
        dist_ref[...] = local_min[:, None]
        idx_ref[...] = local_arg[:, None]

    @pl.when(j > 0)
    def _update():
        prev = dist_ref[:, 0]
        better = local_min < prev
        dist_ref[...] = jnp.where(better, local_min, prev)[:, None]
        idx_ref[...] = jnp.where(better, local_arg, idx_ref[:, 0])[:, None]


@jax.jit
def kernel(queries, keys):
    q_count, d = queries.shape
    k_count = keys.shape[0]
    k_pad = ((k_count + _KB - 1) // _KB) * _KB
    keys_p = jnp.pad(keys, ((0, k_pad - k_count), (0, 0)),
                     constant_values=1e4)
    q_sq = jnp.sum(queries * queries, axis=-1, keepdims=True)
    k_sq = jnp.sum(keys_p * keys_p, axis=-1)[None, :]
    keys_p = keys_p * 2.0

    grid = (q_count // _QB, k_pad // _KB)
    dist, idx = pl.pallas_call(
        functools.partial(_nn_body, kb=_KB),
        grid=grid,
        in_specs=[
            pl.BlockSpec((_QB, d), lambda i, j: (i, 0)),
            pl.BlockSpec((_KB, d), lambda i, j: (j, 0)),
            pl.BlockSpec((_QB, 1), lambda i, j: (i, 0)),
            pl.BlockSpec((1, _KB), lambda i, j: (0, j)),
        ],
        out_specs=[
            pl.BlockSpec((_QB, 1), lambda i, j: (i, 0)),
            pl.BlockSpec((_QB, 1), lambda i, j: (i, 0)),
        ],
        out_shape=[
            jax.ShapeDtypeStruct((q_count, 1), jnp.float32),
            jax.ShapeDtypeStruct((q_count, 1), jnp.int32),
        ],
        compiler_params=pltpu.CompilerParams(
            dimension_semantics=("parallel", "arbitrary")),
    )(queries, keys_p, q_sq, k_sq)
    return dist, idx.astype(jnp.int64)


# speedup vs baseline: 1.8848x; 1.0429x over previous
import functools

import jax
import jax.numpy as jnp
from jax.experimental import pallas as pl
from jax.experimental.pallas import tpu as pltpu


_QB = 2048
_KB = 6400


def _nn_body(q_ref, k_ref, qsq_ref, ksq_ref, dist_ref, idx_ref, *, kb):
    j = pl.program_id(1)
    q2 = q_ref[...]
    k = k_ref[...]
    qb = q2.shape[0]
    cross2 = jax.lax.dot_general(
        q2, k, (((1,), (1,)), ((), ())),
        preferred_element_type=jnp.float32)
    qsq = jnp.broadcast_to(qsq_ref[...], (qb, 128))
    ksq = ksq_ref[...]

    run_min = (qsq + ksq[:, 0:128]) - cross2[:, 0:128]
    run_chunk = jnp.zeros((qb, 128), jnp.int32)
    for c in range(1, kb // 128):
        dc = (qsq + ksq[:, c * 128:(c + 1) * 128]) \
            - cross2[:, c * 128:(c + 1) * 128]
        pred = dc < run_min
        run_min = jnp.where(pred, dc, run_min)
        run_chunk = jnp.where(pred, jnp.int32(c), run_chunk)

    lane = jax.lax.broadcasted_iota(jnp.int32, (qb, 128), 1)
    gidx = run_chunk * 128 + lane + j * kb
    local_min = jnp.min(run_min, axis=1)
    local_arg = jnp.min(
        jnp.where(run_min == local_min[:, None], gidx, jnp.int32(2**31 - 1)),
        axis=1)

    @pl.when(j == 0)
    def _init():
        dist_ref[...] = local_min[:, None]
        idx_ref[...] = local_arg[:, None]

    @pl.when(j > 0)
    def _update():
        prev = dist_ref[:, 0]
        better = local_min < prev
        dist_ref[...] = jnp.where(better, local_min, prev)[:, None]
        idx_ref[...] = jnp.where(better, local_arg, idx_ref[:, 0])[:, None]


@jax.jit
def kernel(queries, keys):
    q_count, d = queries.shape
    k_count = keys.shape[0]
    k_pad = ((k_count + _KB - 1) // _KB) * _KB
    q_sq = jnp.sum(queries * queries, axis=-1, keepdims=True)
    k_sq = jnp.sum(keys * keys, axis=-1)[None, :]
    k_sq = jnp.concatenate(
        [k_sq, jnp.full((1, k_pad - k_count), jnp.inf, jnp.float32)], axis=1)
    queries2 = queries * 2.0

    grid = (q_count // _QB, k_pad // _KB)
    dist, idx = pl.pallas_call(
        functools.partial(_nn_body, kb=_KB),
        grid=grid,
        in_specs=[
            pl.BlockSpec((_QB, d), lambda i, j: (i, 0)),
            pl.BlockSpec((_KB, d), lambda i, j: (j, 0)),
            pl.BlockSpec((_QB, 1), lambda i, j: (i, 0)),
            pl.BlockSpec((1, _KB), lambda i, j: (0, j)),
        ],
        out_specs=[
            pl.BlockSpec((_QB, 1), lambda i, j: (i, 0)),
            pl.BlockSpec((_QB, 1), lambda i, j: (i, 0)),
        ],
        out_shape=[
            jax.ShapeDtypeStruct((q_count, 1), jnp.float32),
            jax.ShapeDtypeStruct((q_count, 1), jnp.int32),
        ],
        compiler_params=pltpu.CompilerParams(
            dimension_semantics=("parallel", "arbitrary")),
    )(queries2, keys, q_sq, k_sq)
    return dist, idx.astype(jnp.int64)
